# baseline (device time: 82457 ns/iter reference)
import jax
import jax.numpy as jnp
from jax import lax
from jax.experimental import pallas as pl
from jax.experimental.pallas import tpu as pltpu

N_DEV = 32
B, SQ, DM = 2, 256, 512
DH = 64
H_PER = 4
ROWS = B * SQ
CHUNK = ROWS // N_DEV

import os
DO_RS = os.environ.get("KERNEL_NO_RS") != "1"
DO_AG = os.environ.get("KERNEL_NO_AG") != "1"


def kernel(x, Wq, K_ext, V_ext, Wo):
    my = lax.axis_index("i")
    K = lax.dynamic_slice_in_dim(K_ext, (my // 2) * 8, 8, axis=2)
    V = lax.dynamic_slice_in_dim(V_ext, (my // 2) * 8, 8, axis=2)

    def body(x_ref, wq_ref, k_ref, v_ref, wo_ref, out_ref,
             acc_ref, stage_ref,
             rs_send_sems, rs_recv_sems, ag_send_sems, ag_recv_sems):
        me = lax.axis_index("i")

        stage_ref[pl.ds(me, 1)] = jnp.zeros((1, CHUNK, DM), jnp.float32)

        x2 = x_ref[...].reshape(ROWS, DM)
        q = jnp.dot(x2, wq_ref[...], preferred_element_type=jnp.float32)
        q4 = q.reshape(B, SQ, H_PER, DH)

        qb = lax.broadcasted_iota(jnp.int32, (SQ, SQ), 0) // 64
        kb = lax.broadcasted_iota(jnp.int32, (SQ, SQ), 1) // 64
        mask = (qb == kb) | ((kb % 4) == (qb % 4))

        odd = (lax.rem(me, 2) == 1)
        for b in range(B):
            k8 = k_ref[b]
            v8 = v_ref[b]
            ctxs = []
            for h in range(H_PER):
                qh = q4[b, :, h, :]
                kh = jnp.where(odd, k8[:, 4 + h, :], k8[:, h, :])
                vh = jnp.where(odd, v8[:, 4 + h, :], v8[:, h, :])
                s = lax.dot_general(
                    qh, kh, (((1,), (1,)), ((), ())),
                    preferred_element_type=jnp.float32) * 0.125
                s = jnp.where(mask, s, -1e9)
                w = jnp.exp(s - jnp.max(s, axis=-1, keepdims=True))
                w = w / jnp.sum(w, axis=-1, keepdims=True)
                ctxs.append(jnp.dot(w, vh, preferred_element_type=jnp.float32))
            ctx_flat = jnp.concatenate(ctxs, axis=1)
            pb = jnp.dot(ctx_flat, wo_ref[...],
                         preferred_element_type=jnp.float32)
            acc_ref[pl.ds(b * 16, 16)] = pb.reshape(16, CHUNK, DM)

        for off in range(1, N_DEV if DO_RS else 1):
            t = lax.rem(me + off, N_DEV)
            pltpu.make_async_remote_copy(
                src_ref=acc_ref.at[t],
                dst_ref=stage_ref.at[me],
                send_sem=rs_send_sems.at[t],
                recv_sem=rs_recv_sems.at[me],
                device_id=(t,),
                device_id_type=pl.DeviceIdType.MESH,
            ).start()

        for off in range(1, N_DEV if DO_RS else 1):
            j = lax.rem(me + off, N_DEV)
            pltpu.make_async_remote_copy(
                src_ref=acc_ref.at[j],
                dst_ref=stage_ref.at[j],
                send_sem=rs_send_sems.at[j],
                recv_sem=rs_recv_sems.at[j],
                device_id=(j,),
                device_id_type=pl.DeviceIdType.MESH,
            ).wait_recv()
        red = (acc_ref[pl.ds(me, 1)]
               + jnp.sum(stage_ref[...], axis=0, keepdims=True))
        acc_ref[pl.ds(me, 1)] = red

        for off in range(1, N_DEV if DO_AG else 1):
            t = lax.rem(me + off, N_DEV)
            pltpu.make_async_remote_copy(
                src_ref=acc_ref.at[me],
                dst_ref=acc_ref.at[me],
                send_sem=ag_send_sems.at[t],
                recv_sem=ag_recv_sems.at[me],
                device_id=(t,),
                device_id_type=pl.DeviceIdType.MESH,
            ).start()

        for off in range(1, N_DEV if DO_AG else 1):
            j = lax.rem(me + off, N_DEV)
            pltpu.make_async_remote_copy(
                src_ref=acc_ref.at[j],
                dst_ref=acc_ref.at[j],
                send_sem=ag_send_sems.at[j],
                recv_sem=ag_recv_sems.at[j],
                device_id=(j,),
                device_id_type=pl.DeviceIdType.MESH,
            ).wait_recv()

        for off in range(1, N_DEV):
            t = lax.rem(me + off, N_DEV)
            if DO_RS:
                pltpu.make_async_remote_copy(
                    src_ref=acc_ref.at[t],
                    dst_ref=stage_ref.at[me],
                    send_sem=rs_send_sems.at[t],
                    recv_sem=rs_recv_sems.at[me],
                    device_id=(t,),
                    device_id_type=pl.DeviceIdType.MESH,
                ).wait_send()
            if DO_AG:
                pltpu.make_async_remote_copy(
                    src_ref=acc_ref.at[me],
                    dst_ref=acc_ref.at[me],
                    send_sem=ag_send_sems.at[t],
                    recv_sem=ag_recv_sems.at[me],
                    device_id=(t,),
                    device_id_type=pl.DeviceIdType.MESH,
                ).wait_send()

        out_ref[...] = acc_ref[...].reshape(B, SQ, DM)

    return pl.pallas_call(
        body,
        out_shape=jax.ShapeDtypeStruct((B, SQ, DM), jnp.float32),
        in_specs=[pl.BlockSpec(memory_space=pltpu.VMEM)] * 5,
        out_specs=pl.BlockSpec(memory_space=pltpu.VMEM),
        scratch_shapes=[
            pltpu.VMEM((N_DEV, CHUNK, DM), jnp.float32),
            pltpu.VMEM((N_DEV, CHUNK, DM), jnp.float32),
            pltpu.SemaphoreType.DMA((N_DEV,)),
            pltpu.SemaphoreType.DMA((N_DEV,)),
            pltpu.SemaphoreType.DMA((N_DEV,)),
            pltpu.SemaphoreType.DMA((N_DEV,)),
        ],
    )(x, Wq, K, V, Wo)


# device time: 73878 ns/iter; 1.1161x vs baseline; 1.1161x over previous
import jax
import jax.numpy as jnp
from jax import lax
from jax.experimental import pallas as pl
from jax.experimental.pallas import tpu as pltpu

N_DEV = 32
B, SQ, DM = 2, 256, 512
DH = 64
H_PER = 4
ROWS = B * SQ
CHUNK = ROWS // N_DEV

import os
DO_RS = os.environ.get("KERNEL_NO_RS") != "1"
DO_AG = os.environ.get("KERNEL_NO_AG") != "1"


def kernel(x, Wq, K_ext, V_ext, Wo):
    my = lax.axis_index("i")
    K = lax.dynamic_slice_in_dim(K_ext, (my // 2) * 8, 8, axis=2)
    V = lax.dynamic_slice_in_dim(V_ext, (my // 2) * 8, 8, axis=2)

    def body(x_ref, wq_ref, k_ref, v_ref, wo_ref, out_ref,
             acc_ref, stage_ref,
             rs_send_sems, rs_recv_sems, ag_send_sems, ag_recv_sems):
        me = lax.axis_index("i")

        barrier_sem = pltpu.get_barrier_semaphore()
        for nbr in (lax.rem(me + 1, N_DEV), lax.rem(me + N_DEV - 1, N_DEV)):
            pl.semaphore_signal(barrier_sem, inc=1, device_id=(nbr,),
                                device_id_type=pl.DeviceIdType.MESH)
        pl.semaphore_wait(barrier_sem, 2)

        stage_ref[pl.ds(me, 1)] = jnp.zeros((1, CHUNK, DM), jnp.float32)

        x2 = x_ref[...].reshape(ROWS, DM)
        q = jnp.dot(x2, wq_ref[...], preferred_element_type=jnp.float32)
        q4 = q.reshape(B, SQ, H_PER, DH)

        qb = lax.broadcasted_iota(jnp.int32, (SQ, SQ), 0) // 64
        kb = lax.broadcasted_iota(jnp.int32, (SQ, SQ), 1) // 64
        mask = (qb == kb) | ((kb % 4) == (qb % 4))

        odd = (lax.rem(me, 2) == 1)
        for b in range(B):
            k8 = k_ref[b]
            v8 = v_ref[b]
            ctxs = []
            for h in range(H_PER):
                qh = q4[b, :, h, :]
                kh = jnp.where(odd, k8[:, 4 + h, :], k8[:, h, :])
                vh = jnp.where(odd, v8[:, 4 + h, :], v8[:, h, :])
                s = lax.dot_general(
                    qh, kh, (((1,), (1,)), ((), ())),
                    preferred_element_type=jnp.float32) * 0.125
                s = jnp.where(mask, s, -1e9)
                w = jnp.exp(s - jnp.max(s, axis=-1, keepdims=True))
                w = w / jnp.sum(w, axis=-1, keepdims=True)
                ctxs.append(jnp.dot(w, vh, preferred_element_type=jnp.float32))
            ctx_flat = jnp.concatenate(ctxs, axis=1)
            pb = jnp.dot(ctx_flat, wo_ref[...],
                         preferred_element_type=jnp.float32)
            acc_ref[pl.ds(b * 16, 16)] = pb.reshape(16, CHUNK, DM)

            for off in range(1, N_DEV if DO_RS else 1):
                t = lax.rem(me + off, N_DEV)

                @pl.when((t >= b * 16) & (t < (b + 1) * 16))
                def _send(t=t):
                    pltpu.make_async_remote_copy(
                        src_ref=acc_ref.at[t],
                        dst_ref=stage_ref.at[me],
                        send_sem=rs_send_sems.at[t],
                        recv_sem=rs_recv_sems.at[me],
                        device_id=(t,),
                        device_id_type=pl.DeviceIdType.MESH,
                    ).start()

        for off in range(1, N_DEV if DO_RS else 1):
            j = lax.rem(me + off, N_DEV)
            pltpu.make_async_remote_copy(
                src_ref=acc_ref.at[j],
                dst_ref=stage_ref.at[j],
                send_sem=rs_send_sems.at[j],
                recv_sem=rs_recv_sems.at[j],
                device_id=(j,),
                device_id_type=pl.DeviceIdType.MESH,
            ).wait_recv()
        red = (acc_ref[pl.ds(me, 1)]
               + jnp.sum(stage_ref[...], axis=0, keepdims=True))
        acc_ref[pl.ds(me, 1)] = red

        for off in range(1, N_DEV if DO_AG else 1):
            t = lax.rem(me + off, N_DEV)
            pltpu.make_async_remote_copy(
                src_ref=acc_ref.at[me],
                dst_ref=acc_ref.at[me],
                send_sem=ag_send_sems.at[t],
                recv_sem=ag_recv_sems.at[me],
                device_id=(t,),
                device_id_type=pl.DeviceIdType.MESH,
            ).start()

        for off in range(1, N_DEV if DO_AG else 1):
            j = lax.rem(me + off, N_DEV)
            pltpu.make_async_remote_copy(
                src_ref=acc_ref.at[j],
                dst_ref=acc_ref.at[j],
                send_sem=ag_send_sems.at[j],
                recv_sem=ag_recv_sems.at[j],
                device_id=(j,),
                device_id_type=pl.DeviceIdType.MESH,
            ).wait_recv()

        for off in range(1, N_DEV):
            t = lax.rem(me + off, N_DEV)
            if DO_RS:
                pltpu.make_async_remote_copy(
                    src_ref=acc_ref.at[t],
                    dst_ref=stage_ref.at[me],
                    send_sem=rs_send_sems.at[t],
                    recv_sem=rs_recv_sems.at[me],
                    device_id=(t,),
                    device_id_type=pl.DeviceIdType.MESH,
                ).wait_send()
            if DO_AG:
                pltpu.make_async_remote_copy(
                    src_ref=acc_ref.at[me],
                    dst_ref=acc_ref.at[me],
                    send_sem=ag_send_sems.at[t],
                    recv_sem=ag_recv_sems.at[me],
                    device_id=(t,),
                    device_id_type=pl.DeviceIdType.MESH,
                ).wait_send()

        out_ref[...] = acc_ref[...].reshape(B, SQ, DM)

    return pl.pallas_call(
        body,
        out_shape=jax.ShapeDtypeStruct((B, SQ, DM), jnp.float32),
        in_specs=[pl.BlockSpec(memory_space=pltpu.VMEM)] * 5,
        out_specs=pl.BlockSpec(memory_space=pltpu.VMEM),
        scratch_shapes=[
            pltpu.VMEM((N_DEV, CHUNK, DM), jnp.float32),
            pltpu.VMEM((N_DEV, CHUNK, DM), jnp.float32),
            pltpu.SemaphoreType.DMA((N_DEV,)),
            pltpu.SemaphoreType.DMA((N_DEV,)),
            pltpu.SemaphoreType.DMA((N_DEV,)),
            pltpu.SemaphoreType.DMA((N_DEV,)),
        ],
        compiler_params=pltpu.CompilerParams(collective_id=0),
    )(x, Wq, K, V, Wo)
